# baseline (speedup 1.0000x reference)
import functools

import jax
import jax.numpy as jnp
from jax import lax
from jax.experimental import pallas as pl
from jax.experimental.pallas import tpu as pltpu
from jax.experimental.pallas import tpu_sc as plsc

V = 1_000_000
B = 16384
T = 200
L = 16
NS = 16
NC = 2
CPT = B // NS
G = CPT // 128
R = 40


RB = 8
NBLK = T // RB
NRING = 2


def _body(label_h, text_h, w0_h, w1_h, out0_h, out1_h, lc0_h, lc1_h,
          hist_sh, lcsum_sh, labels_v, vals_v, vals8_v, accf_v, idx0_v,
          lcf_v, tile_bufs, idx_bufs, lsems, ssems):
    c = lax.axis_index("c")
    s = lax.axis_index("s")

    @pl.when(jnp.logical_and(s == 0, c == 0))
    def _():
        pltpu.sync_copy(w0_h, hist_sh)

    @pl.when(jnp.logical_and(s == 0, c == 1))
    def _():
        pltpu.sync_copy(w1_h, hist_sh)

    pltpu.sync_copy(label_h.at[pl.ds(s * CPT, CPT)], labels_v)
    accf = jnp.zeros((L,), jnp.float32)
    for g in range(G):
        for k in range(128 // L):
            lv = labels_v[pl.ds(g * 128 + k * L, L)]
            mv = jnp.where(lv == c, 1.0, 0.0).astype(jnp.float32)
            vals_v[pl.ds(g * 128 + k * L, L)] = mv
            accf = accf + mv
    accf_v[...] = accf
    idx0_v[...] = jnp.zeros((L,), jnp.int32)

    def vrep(g, cy):
        mv = vals_v[pl.ds(g * L, L)]
        for r in range(RB):
            vals8_v[pl.ds(r * CPT + g * L, L)] = mv
        return cy
    lax.fori_loop(0, CPT // L, vrep, 0)

    @pl.when(s == 1)
    def _():
        lcf_v[...] = jnp.zeros((L,), jnp.float32)
        pltpu.sync_copy(lcf_v, lcsum_sh)

    plsc.subcore_barrier()

    pltpu.sync_copy(accf_v, lcsum_sh.at[idx0_v], add=True)

    def load_block(bi, j):
        r0 = pl.multiple_of(bi * RB, 8)
        for t in range(G):
            ct = pl.multiple_of(s * CPT + t * 128, 128)
            pltpu.async_copy(text_h.at[pl.ds(r0, RB), pl.ds(ct, 128)],
                             tile_bufs[j][t], lsems[j])

    def process_block(bi, j, drain, next_load):
        for t in range(G):
            pltpu.make_async_copy(text_h.at[pl.ds(0, RB), pl.ds(0, 128)],
                                  tile_bufs[j][t], lsems[j]).wait()

        if drain is not None:
            @pl.when(drain)
            def _():
                pltpu.make_async_copy(vals8_v, hist_sh.at[idx_bufs[j]],
                                      ssems[j]).wait()

        def rp(k, cy):
            for r in range(RB):
                for t in range(G):
                    idx_bufs[j][pl.ds(r * CPT + t * 128 + k * L, L)] = \
                        tile_bufs[j][t][r, pl.ds(k * L, L)]
            return cy
        lax.fori_loop(0, 128 // L, rp, 0)

        pltpu.async_copy(vals8_v, hist_sh.at[idx_bufs[j]], ssems[j],
                         add=True)

        if next_load is not None:
            @pl.when(next_load)
            def _():
                load_block(bi + NRING, j)

    load_block(0, 0)
    load_block(1, 1)

    def round_(ob, carry):
        for jb in range(NRING):
            bi = ob * NRING + jb
            process_block(bi, jb, drain=ob >= 1, next_load=bi < NBLK - NRING)
        return carry

    lax.fori_loop(0, (NBLK - 1) // NRING, round_, 0)
    process_block(NBLK - 1, 0, drain=jnp.bool_(True), next_load=None)

    for j in range(NRING):
        pltpu.make_async_copy(vals8_v, hist_sh.at[idx_bufs[j]],
                              ssems[j]).wait()

    plsc.subcore_barrier()

    @pl.when(jnp.logical_and(s == 0, c == 0))
    def _():
        pltpu.sync_copy(hist_sh, out0_h)

    @pl.when(jnp.logical_and(s == 0, c == 1))
    def _():
        pltpu.sync_copy(hist_sh, out1_h)

    @pl.when(jnp.logical_and(s == 1, c == 0))
    def _():
        pltpu.sync_copy(lcsum_sh, lc0_h)

    @pl.when(jnp.logical_and(s == 1, c == 1))
    def _():
        pltpu.sync_copy(lcsum_sh, lc1_h)


_hist = functools.partial(
    pl.kernel,
    out_type=[
        jax.ShapeDtypeStruct((V,), jnp.float32),
        jax.ShapeDtypeStruct((V,), jnp.float32),
        jax.ShapeDtypeStruct((L,), jnp.float32),
        jax.ShapeDtypeStruct((L,), jnp.float32),
    ],
    mesh=plsc.VectorSubcoreMesh(core_axis_name="c", subcore_axis_name="s"),
    scratch_types=[
        pltpu.VMEM_SHARED((V,), jnp.float32),
        pltpu.VMEM_SHARED((L,), jnp.float32),
        pltpu.VMEM((CPT,), jnp.int32),
        pltpu.VMEM((CPT,), jnp.float32),
        pltpu.VMEM((RB * CPT,), jnp.float32),
        pltpu.VMEM((L,), jnp.float32),
        pltpu.VMEM((L,), jnp.int32),
        pltpu.VMEM((L,), jnp.float32),
        [[pltpu.VMEM((RB, 128), jnp.int32)
          for _ in range(G)] for _ in range(NRING)],
        [pltpu.VMEM((RB * CPT,), jnp.int32)
         for _ in range(NRING)],
        [pltpu.SemaphoreType.DMA] * NRING,
        [pltpu.SemaphoreType.DMA] * NRING,
    ],
)(_body)


def kernel(label, text, w_counts0, w_counts1):
    w0, w1, lc0v, lc1v = _hist(label.astype(jnp.int32),
                               text.astype(jnp.int32),
                               w_counts0, w_counts1)
    return w0, w1, lc0v[0].astype(jnp.int32), lc1v[0].astype(jnp.int32)


# speedup vs baseline: 1.0079x; 1.0079x over previous
import functools

import jax
import jax.numpy as jnp
from jax import lax
from jax.experimental import pallas as pl
from jax.experimental.pallas import tpu as pltpu
from jax.experimental.pallas import tpu_sc as plsc

V = 1_000_000
B = 16384
T = 200
L = 16
NS = 16
NC = 2
CPT = B // NS
G = CPT // 128
RB = 8
NBLK = T // RB
NRING = 2


def _body(label_h, text_h, w0_h, w1_h, out0_h, out1_h, lc0_h, lc1_h,
          hist_sh, lcsum_sh, labels_v, vals_v, accf_v, idx0_v, lcf_v,
          tile_bufs, idx_bufs, lsems, ssems):
    c = lax.axis_index("c")
    s = lax.axis_index("s")

    @pl.when(jnp.logical_and(s == 0, c == 0))
    def _():
        pltpu.sync_copy(w0_h, hist_sh)

    @pl.when(jnp.logical_and(s == 0, c == 1))
    def _():
        pltpu.sync_copy(w1_h, hist_sh)

    pltpu.sync_copy(label_h.at[pl.ds(s * CPT, CPT)], labels_v)
    accf = jnp.zeros((L,), jnp.float32)
    for g in range(G):
        for k in range(128 // L):
            lv = labels_v[pl.ds(g * 128 + k * L, L)]
            mv = jnp.where(lv == c, 1.0, 0.0).astype(jnp.float32)
            vals_v[pl.ds(g * 128 + k * L, L)] = mv
            accf = accf + mv
    accf_v[...] = accf
    idx0_v[...] = jnp.zeros((L,), jnp.int32)

    @pl.when(s == 1)
    def _():
        lcf_v[...] = jnp.zeros((L,), jnp.float32)
        pltpu.sync_copy(lcf_v, lcsum_sh)

    plsc.subcore_barrier()

    pltpu.sync_copy(accf_v, lcsum_sh.at[idx0_v], add=True)

    def load_block(bi, j):
        r0 = pl.multiple_of(bi * RB, 8)
        for t in range(G):
            ct = pl.multiple_of(s * CPT + t * 128, 128)
            pltpu.async_copy(text_h.at[pl.ds(r0, RB), pl.ds(ct, 128)],
                             tile_bufs[j][t], lsems[j])

    def process_block(bi, j, drain, next_load):
        for t in range(G):
            pltpu.make_async_copy(text_h.at[pl.ds(0, RB), pl.ds(0, 128)],
                                  tile_bufs[j][t], lsems[j]).wait()

        if drain is not None:
            @pl.when(drain)
            def _():
                for r in range(RB):
                    pltpu.make_async_copy(vals_v, hist_sh.at[idx_bufs[j][0]],
                                          ssems[j]).wait()

        def rp(k, cy):
            for r in range(RB):
                for t in range(G):
                    idx_bufs[j][r][pl.ds(t * 128 + k * L, L)] = \
                        tile_bufs[j][t][r, pl.ds(k * L, L)]
            return cy
        lax.fori_loop(0, 128 // L, rp, 0)

        for r in range(RB):
            pltpu.async_copy(vals_v, hist_sh.at[idx_bufs[j][r]],
                             ssems[j], add=True)

        if next_load is not None:
            @pl.when(next_load)
            def _():
                load_block(bi + NRING, j)

    load_block(0, 0)
    load_block(1, 1)

    def round_(ob, carry):
        for jb in range(NRING):
            bi = ob * NRING + jb
            process_block(bi, jb, drain=ob >= 1, next_load=bi < NBLK - NRING)
        return carry

    lax.fori_loop(0, (NBLK - 1) // NRING, round_, 0)
    process_block(NBLK - 1, 0, drain=jnp.bool_(True), next_load=None)

    for j in range(NRING):
        for r in range(RB):
            pltpu.make_async_copy(vals_v, hist_sh.at[idx_bufs[j][0]],
                                  ssems[j]).wait()

    plsc.subcore_barrier()

    @pl.when(jnp.logical_and(s == 0, c == 0))
    def _():
        pltpu.sync_copy(hist_sh, out0_h)

    @pl.when(jnp.logical_and(s == 0, c == 1))
    def _():
        pltpu.sync_copy(hist_sh, out1_h)

    @pl.when(jnp.logical_and(s == 1, c == 0))
    def _():
        pltpu.sync_copy(lcsum_sh, lc0_h)

    @pl.when(jnp.logical_and(s == 1, c == 1))
    def _():
        pltpu.sync_copy(lcsum_sh, lc1_h)


_hist = functools.partial(
    pl.kernel,
    out_type=[
        jax.ShapeDtypeStruct((V,), jnp.float32),
        jax.ShapeDtypeStruct((V,), jnp.float32),
        jax.ShapeDtypeStruct((L,), jnp.float32),
        jax.ShapeDtypeStruct((L,), jnp.float32),
    ],
    mesh=plsc.VectorSubcoreMesh(core_axis_name="c", subcore_axis_name="s"),
    scratch_types=[
        pltpu.VMEM_SHARED((V,), jnp.float32),
        pltpu.VMEM_SHARED((L,), jnp.float32),
        pltpu.VMEM((CPT,), jnp.int32),
        pltpu.VMEM((CPT,), jnp.float32),
        pltpu.VMEM((L,), jnp.float32),
        pltpu.VMEM((L,), jnp.int32),
        pltpu.VMEM((L,), jnp.float32),
        [[pltpu.VMEM((RB, 128), jnp.int32)
          for _ in range(G)] for _ in range(NRING)],
        [[pltpu.VMEM((CPT,), jnp.int32)
          for _ in range(RB)] for _ in range(NRING)],
        [pltpu.SemaphoreType.DMA] * NRING,
        [pltpu.SemaphoreType.DMA] * NRING,
    ],
)(_body)


def kernel(label, text, w_counts0, w_counts1):
    w0, w1, lc0v, lc1v = _hist(label.astype(jnp.int32),
                               text.astype(jnp.int32),
                               w_counts0, w_counts1)
    return w0, w1, lc0v[0].astype(jnp.int32), lc1v[0].astype(jnp.int32)
